# Initial kernel scaffold; baseline (speedup 1.0000x reference)
import jax
import jax.numpy as jnp
from jax.experimental import pallas as pl


def kernel(x, ids, ratings_emd, id_emd, W, b):
    raise NotImplementedError("write your pallas kernel here")


# speedup vs baseline: 2.3693x; 2.3693x over previous
import functools

import jax
import jax.numpy as jnp
from jax import lax
from jax.experimental import pallas as pl
from jax.experimental.pallas import tpu as pltpu
from jax.experimental.pallas import tpu_sc as plsc

INPUT = 64
HID = 128
BATCH = 4096
SEQ = 200
N_TOK = BATCH * SEQ
CHUNK = 128
N_CHUNKS = N_TOK // CHUNK
NW = 32
CHUNKS_PER_W = N_CHUNKS // NW
RING = 8
N_GROUPS = CHUNKS_PER_W // RING

BB = 64


def _sc_gather(table, idx2d):
  mesh = plsc.VectorSubcoreMesh(core_axis_name="c", subcore_axis_name="s")

  @functools.partial(
      pl.kernel,
      out_type=jax.ShapeDtypeStruct((N_CHUNKS, CHUNK, INPUT), jnp.float32),
      mesh=mesh,
      scratch_types=[
          pltpu.VMEM((CHUNKS_PER_W, CHUNK), jnp.int32),
          pltpu.VMEM((RING, CHUNK, INPUT), jnp.float32),
          pltpu.SemaphoreType.DMA,
          pltpu.SemaphoreType.DMA,
      ],
      compiler_params=pltpu.CompilerParams(use_tc_tiling_on_sc=False),
  )
  def k(table_hbm, idx_hbm, out_hbm, idx_v, buf_v, gsem, wsem):
    wid = lax.axis_index("s") * 2 + lax.axis_index("c")
    base = wid * CHUNKS_PER_W
    pltpu.sync_copy(idx_hbm.at[pl.ds(base, CHUNKS_PER_W)], idx_v)

    def group(g, _):
      @pl.when(g > 0)
      def _():
        for r in range(RING):
          pltpu.make_async_copy(
              buf_v.at[r], out_hbm.at[base], wsem).wait()
      handles = []
      for r in range(RING):
        handles.append(pltpu.make_async_copy(
            table_hbm.at[idx_v.at[g * RING + r]], buf_v.at[r], gsem))
        handles[-1].start()
      for h in handles:
        h.wait()
      for r in range(RING):
        pltpu.make_async_copy(
            buf_v.at[r], out_hbm.at[base + g * RING + r], wsem).start()
      return 0

    lax.fori_loop(0, N_GROUPS, group, 0)
    for r in range(RING):
      pltpu.make_async_copy(buf_v.at[r], out_hbm.at[base], wsem).wait()

  return k(table, idx2d)


def _tc_proj(gathered, oh, seg, rp, wid_t, interpret=False):
  def body(oh_ref, g_ref, seg_ref, rp_ref, w_ref, o_ref):
    g = g_ref[...]
    h = jnp.dot(g, w_ref[...], preferred_element_type=jnp.float32)
    oh = oh_ref[...].astype(jnp.float32)
    h = h + jnp.dot(oh, rp_ref[...], preferred_element_type=jnp.float32)
    h = jnp.maximum(h, 0.0)
    o_ref[...] = jnp.dot(seg_ref[...], h, preferred_element_type=jnp.float32)

  return pl.pallas_call(
      body,
      grid=(BATCH // BB,),
      in_specs=[
          pl.BlockSpec((BB * SEQ, 8), lambda i: (i, 0)),
          pl.BlockSpec((BB * SEQ, INPUT), lambda i: (i, 0)),
          pl.BlockSpec((BB, BB * SEQ), lambda i: (0, 0)),
          pl.BlockSpec((8, HID), lambda i: (0, 0)),
          pl.BlockSpec((INPUT, HID), lambda i: (0, 0)),
      ],
      out_specs=pl.BlockSpec((BB, HID), lambda i: (i, 0)),
      out_shape=jax.ShapeDtypeStruct((BATCH, HID), jnp.float32),
      interpret=interpret,
  )(oh, gathered, seg, rp, wid_t)


def kernel(x, ids, ratings_emd, id_emd, W, b):
  rp = ratings_emd @ W[:, :INPUT].T + b
  rp = jnp.concatenate([rp, jnp.zeros((2, HID), jnp.float32)], axis=0)
  wid_t = W[:, INPUT:].T
  seg = (lax.broadcasted_iota(jnp.int32, (BB, BB * SEQ), 1) // SEQ ==
         lax.broadcasted_iota(jnp.int32, (BB, BB * SEQ), 0)
         ).astype(jnp.float32) * (1.0 / SEQ)
  oh = (x.reshape(N_TOK, 1) == jnp.arange(8, dtype=jnp.int32)
        ).astype(jnp.int8)
  idx2d = ids.reshape(N_CHUNKS, CHUNK)
  gathered = _sc_gather(id_emd, idx2d)
  g_flat = gathered.reshape(N_TOK, INPUT)
  return _tc_proj(g_flat, oh, seg, rp, wid_t)


# speedup vs baseline: 2.9970x; 1.2649x over previous
import functools

import jax
import jax.numpy as jnp
from jax import lax
from jax.experimental import pallas as pl
from jax.experimental.pallas import tpu as pltpu
from jax.experimental.pallas import tpu_sc as plsc

INPUT = 64
HID = 128
BATCH = 4096
SEQ = 200
N_TOK = BATCH * SEQ
CHUNK = 128
N_CHUNKS = N_TOK // CHUNK
NW = 32
CHUNKS_PER_W = N_CHUNKS // NW
RING = 8
N_GROUPS = CHUNKS_PER_W // RING

BB = 64
PAIRS = N_TOK // 2
BPAIR = BB * SEQ // 2


def _sc_gather(table, idx2d):
  mesh = plsc.VectorSubcoreMesh(core_axis_name="c", subcore_axis_name="s")

  @functools.partial(
      pl.kernel,
      out_type=jax.ShapeDtypeStruct((N_CHUNKS, CHUNK, INPUT), jnp.float32),
      mesh=mesh,
      scratch_types=[
          pltpu.VMEM((CHUNKS_PER_W, CHUNK), jnp.int32),
          pltpu.VMEM((RING, CHUNK, INPUT), jnp.float32),
          pltpu.SemaphoreType.DMA,
          pltpu.SemaphoreType.DMA,
      ],
      compiler_params=pltpu.CompilerParams(use_tc_tiling_on_sc=False),
  )
  def k(table_hbm, idx_hbm, out_hbm, idx_v, buf_v, gsem, wsem):
    wid = lax.axis_index("s") * 2 + lax.axis_index("c")
    base = wid * CHUNKS_PER_W
    pltpu.sync_copy(idx_hbm.at[pl.ds(base, CHUNKS_PER_W)], idx_v)

    def group(g, _):
      @pl.when(g > 0)
      def _():
        for r in range(RING):
          pltpu.make_async_copy(
              buf_v.at[r], out_hbm.at[base], wsem).wait()
      handles = []
      for r in range(RING):
        handles.append(pltpu.make_async_copy(
            table_hbm.at[idx_v.at[g * RING + r]], buf_v.at[r], gsem))
        handles[-1].start()
      for h in handles:
        h.wait()
      for r in range(RING):
        pltpu.make_async_copy(
            buf_v.at[r], out_hbm.at[base + g * RING + r], wsem).start()
      return 0

    lax.fori_loop(0, N_GROUPS, group, 0)
    for r in range(RING):
      pltpu.make_async_copy(
          buf_v.at[r], out_hbm.at[base], wsem).wait()

  return k(table, idx2d)


def _tc_proj(g2, oh2, seg2, rp2b, w2b, interpret=False):
  def body(oh_ref, g_ref, seg_ref, rp_ref, w_ref, o_ref):
    g = g_ref[...]
    h = jnp.dot(g, w_ref[...], preferred_element_type=jnp.float32)
    oh = oh_ref[...].astype(jnp.float32)
    h = h + jnp.dot(oh, rp_ref[...], preferred_element_type=jnp.float32)
    h = jnp.maximum(h, 0.0)
    hsum = h[:, :HID] + h[:, HID:]
    o_ref[...] = jnp.dot(seg_ref[...], hsum,
                         preferred_element_type=jnp.float32)

  return pl.pallas_call(
      body,
      grid=(BATCH // BB,),
      in_specs=[
          pl.BlockSpec((BPAIR, 16), lambda i: (i, 0)),
          pl.BlockSpec((BPAIR, 2 * INPUT), lambda i: (i, 0)),
          pl.BlockSpec((BB, BPAIR), lambda i: (0, 0)),
          pl.BlockSpec((16, 2 * HID), lambda i: (0, 0)),
          pl.BlockSpec((2 * INPUT, 2 * HID), lambda i: (0, 0)),
      ],
      out_specs=pl.BlockSpec((BB, HID), lambda i: (i, 0)),
      out_shape=jax.ShapeDtypeStruct((BATCH, HID), jnp.float32),
      interpret=interpret,
  )(oh2, g2, seg2, rp2b, w2b)


def kernel(x, ids, ratings_emd, id_emd, W, b):
  rp = ratings_emd @ W[:, :INPUT].T + b
  rp = jnp.concatenate([rp, jnp.zeros((2, HID), jnp.float32)], axis=0)
  wid_t = W[:, INPUT:].T
  zz = jnp.zeros((INPUT, HID), jnp.float32)
  w2b = jnp.block([[wid_t, zz], [zz, wid_t]])
  z8 = jnp.zeros((8, HID), jnp.float32)
  rp2b = jnp.block([[rp, z8], [z8, rp]])
  seg2 = (lax.broadcasted_iota(jnp.int32, (BB, BPAIR), 1) // (SEQ // 2) ==
          lax.broadcasted_iota(jnp.int32, (BB, BPAIR), 0)
          ).astype(jnp.float32) * (1.0 / SEQ)
  oh2 = (x.reshape(PAIRS, 2, 1) == jnp.arange(8, dtype=jnp.int32)
         ).reshape(PAIRS, 16).astype(jnp.int8)
  idx2d = ids.reshape(N_CHUNKS, CHUNK)
  gathered = _sc_gather(id_emd, idx2d)
  g2 = gathered.reshape(PAIRS, 2 * INPUT)
  return _tc_proj(g2, oh2, seg2, rp2b, w2b)


# speedup vs baseline: 3.1063x; 1.0365x over previous
import functools

import jax
import jax.numpy as jnp
from jax import lax
from jax.experimental import pallas as pl
from jax.experimental.pallas import tpu as pltpu
from jax.experimental.pallas import tpu_sc as plsc

INPUT = 64
HID = 128
BATCH = 4096
SEQ = 200
N_TOK = BATCH * SEQ
CHUNK = 128
N_CHUNKS = N_TOK // CHUNK
NW = 32
CHUNKS_PER_W = N_CHUNKS // NW
RING = 4
N_GROUPS = CHUNKS_PER_W // RING
REP = 1024

BB = 64
PAIRS = N_TOK // 2
BPAIR = BB * SEQ // 2


def _sc_gather(table_id, table_rt, idx2d, idxr2d):
  mesh = plsc.VectorSubcoreMesh(core_axis_name="c", subcore_axis_name="s")

  @functools.partial(
      pl.kernel,
      out_type=(
          jax.ShapeDtypeStruct((N_CHUNKS, CHUNK, INPUT), jnp.float32),
          jax.ShapeDtypeStruct((N_CHUNKS, CHUNK, INPUT), jnp.float32),
      ),
      mesh=mesh,
      scratch_types=[
          pltpu.VMEM((CHUNKS_PER_W, CHUNK), jnp.int32),
          pltpu.VMEM((CHUNKS_PER_W, CHUNK), jnp.int32),
          pltpu.VMEM((RING, CHUNK, INPUT), jnp.float32),
          pltpu.VMEM((RING, CHUNK, INPUT), jnp.float32),
          pltpu.SemaphoreType.DMA,
          pltpu.SemaphoreType.DMA,
      ],
      compiler_params=pltpu.CompilerParams(use_tc_tiling_on_sc=False),
  )
  def k(tid_hbm, trt_hbm, idx_hbm, idxr_hbm, oid_hbm, ort_hbm,
        idx_v, idxr_v, bid_v, brt_v, gsem, wsem):
    wid = lax.axis_index("s") * 2 + lax.axis_index("c")
    base = wid * CHUNKS_PER_W
    pltpu.sync_copy(idx_hbm.at[pl.ds(base, CHUNKS_PER_W)], idx_v)
    pltpu.sync_copy(idxr_hbm.at[pl.ds(base, CHUNKS_PER_W)], idxr_v)

    def group(g, _):
      @pl.when(g > 0)
      def _():
        for r in range(RING):
          pltpu.make_async_copy(bid_v.at[r], oid_hbm.at[base], wsem).wait()
          pltpu.make_async_copy(brt_v.at[r], ort_hbm.at[base], wsem).wait()
      handles = []
      for r in range(RING):
        c = g * RING + r
        handles.append(pltpu.make_async_copy(
            tid_hbm.at[idx_v.at[c]], bid_v.at[r], gsem))
        handles.append(pltpu.make_async_copy(
            trt_hbm.at[idxr_v.at[c]], brt_v.at[r], gsem))
      for h in handles:
        h.start()
      for h in handles:
        h.wait()
      for r in range(RING):
        c = g * RING + r
        pltpu.make_async_copy(bid_v.at[r], oid_hbm.at[base + c], wsem).start()
        pltpu.make_async_copy(brt_v.at[r], ort_hbm.at[base + c], wsem).start()
      return 0

    lax.fori_loop(0, N_GROUPS, group, 0)
    for r in range(RING):
      pltpu.make_async_copy(bid_v.at[r], oid_hbm.at[base], wsem).wait()
      pltpu.make_async_copy(brt_v.at[r], ort_hbm.at[base], wsem).wait()

  return k(table_id, table_rt, idx2d, idxr2d)


def _tc_proj(gid2, grt2, seg2, b2, wid2b, wrt2b, interpret=False):
  def body(gid_ref, grt_ref, seg_ref, b_ref, wid_ref, wrt_ref, o_ref):
    h = jnp.dot(gid_ref[...], wid_ref[...],
                preferred_element_type=jnp.float32)
    h = h + jnp.dot(grt_ref[...], wrt_ref[...],
                    preferred_element_type=jnp.float32)
    h = h + b_ref[0:1, :]
    h = jnp.maximum(h, 0.0)
    hsum = h[:, :HID] + h[:, HID:]
    o_ref[...] = jnp.dot(seg_ref[...], hsum,
                         preferred_element_type=jnp.float32)

  return pl.pallas_call(
      body,
      grid=(BATCH // BB,),
      in_specs=[
          pl.BlockSpec((BPAIR, 2 * INPUT), lambda i: (i, 0)),
          pl.BlockSpec((BPAIR, 2 * INPUT), lambda i: (i, 0)),
          pl.BlockSpec((BB, BPAIR), lambda i: (0, 0)),
          pl.BlockSpec((8, 2 * HID), lambda i: (0, 0)),
          pl.BlockSpec((2 * INPUT, 2 * HID), lambda i: (0, 0)),
          pl.BlockSpec((2 * INPUT, 2 * HID), lambda i: (0, 0)),
      ],
      out_specs=pl.BlockSpec((BB, HID), lambda i: (i, 0)),
      out_shape=jax.ShapeDtypeStruct((BATCH, HID), jnp.float32),
      interpret=interpret,
  )(gid2, grt2, seg2, b2, wid2b, wrt2b)


def kernel(x, ids, ratings_emd, id_emd, W, b):
  wrt_t = W[:, :INPUT].T
  wid_t = W[:, INPUT:].T
  zz = jnp.zeros((INPUT, HID), jnp.float32)
  wid2b = jnp.block([[wid_t, zz], [zz, wid_t]])
  wrt2b = jnp.block([[wrt_t, zz], [zz, wrt_t]])
  b2 = jnp.broadcast_to(jnp.concatenate([b, b])[None, :], (8, 2 * HID))
  seg2 = (lax.broadcasted_iota(jnp.int32, (BB, BPAIR), 1) // (SEQ // 2) ==
          lax.broadcasted_iota(jnp.int32, (BB, BPAIR), 0)
          ).astype(jnp.float32) * (1.0 / SEQ)
  table_rt = jnp.repeat(ratings_emd, REP, axis=0)
  idx2d = ids.reshape(N_CHUNKS, CHUNK)
  x2d = x.reshape(N_CHUNKS, CHUNK)
  pos2d = (lax.broadcasted_iota(jnp.int32, (N_CHUNKS, CHUNK), 0) % 8) * CHUNK \
      + lax.broadcasted_iota(jnp.int32, (N_CHUNKS, CHUNK), 1)
  idxr2d = x2d * REP + pos2d
  gid, grt = _sc_gather(id_emd, table_rt, idx2d, idxr2d)
  gid2 = gid.reshape(PAIRS, 2 * INPUT)
  grt2 = grt.reshape(PAIRS, 2 * INPUT)
  return _tc_proj(gid2, grt2, seg2, b2, wid2b, wrt2b)
